# Initial kernel scaffold; baseline (speedup 1.0000x reference)
import jax
import jax.numpy as jnp
from jax.experimental import pallas as pl


def kernel(x, item_emb, pos_emb):
    raise NotImplementedError("write your pallas kernel here")


# speedup vs baseline: 3.1461x; 3.1461x over previous
import functools

import jax
import jax.numpy as jnp
from jax import lax
from jax.experimental import pallas as pl
from jax.experimental.pallas import tpu as pltpu
from jax.experimental.pallas import tpu_sc as plsc

BATCH = 4096
SEQ = 200
D = 32
NUM_ROWS = BATCH * SEQ
NC, NS = 2, 16
NW = NC * NS
RPW = NUM_ROWS // NW
CH = 1600
NCHUNK = RPW // CH
GW = 100
NG = CH // GW

_mesh = plsc.VectorSubcoreMesh(core_axis_name="c", subcore_axis_name="s")


@functools.partial(
    pl.kernel,
    mesh=_mesh,
    out_type=jax.ShapeDtypeStruct((NUM_ROWS, D), jnp.float32),
    scratch_types=[
        pltpu.VMEM((NG, GW), jnp.int32),
        pltpu.VMEM((CH, D), jnp.float32),
        pltpu.VMEM((SEQ, D), jnp.float32),
        pltpu.SemaphoreType.DMA,
    ],
    compiler_params=pltpu.CompilerParams(use_tc_tiling_on_sc=False),
)
def _emb_lookup(x_hbm, item_hbm, pos_hbm, out_hbm, idx_v, rows_v, pos_v, sem):
    wid = lax.axis_index("s") * NC + lax.axis_index("c")
    pltpu.sync_copy(pos_hbm, pos_v)
    base_row = wid * RPW

    def chunk_body(c, carry):
        r0 = pl.multiple_of(base_row + c * CH, CH)
        pltpu.sync_copy(x_hbm.at[pl.ds(pl.multiple_of(r0 // GW, NG), NG)], idx_v)
        copies = [
            pltpu.async_copy(item_hbm.at[idx_v.at[j]],
                             rows_v.at[pl.ds(j * GW, GW)], sem)
            for j in range(NG)
        ]
        for cp in copies:
            cp.wait()

        def row_body(r, rcarry):
            p0 = pos_v[r, pl.ds(0, 16)]
            p1 = pos_v[r, pl.ds(16, 16)]
            for k in range(CH // SEQ):
                plsc.addupdate(rows_v.at[k * SEQ + r, pl.ds(0, 16)], p0)
                plsc.addupdate(rows_v.at[k * SEQ + r, pl.ds(16, 16)], p1)
            return rcarry

        lax.fori_loop(0, SEQ, row_body, 0)
        pltpu.sync_copy(rows_v, out_hbm.at[pl.ds(r0, CH)])
        return carry

    lax.fori_loop(0, NCHUNK, chunk_body, 0)


def kernel(x, item_emb, pos_emb):
    xf = x.reshape(NUM_ROWS // GW, GW)
    out = _emb_lookup(xf, item_emb, pos_emb)
    return out.reshape(BATCH, SEQ, D)


# speedup vs baseline: 3.1559x; 1.0031x over previous
import functools

import jax
import jax.numpy as jnp
from jax import lax
from jax.experimental import pallas as pl
from jax.experimental.pallas import tpu as pltpu
from jax.experimental.pallas import tpu_sc as plsc

BATCH = 4096
SEQ = 200
D = 32
NC, NS = 2, 16
NW = NC * NS
BPW = BATCH // NW
CB = 8
NCHUNK = BPW // CB
GW = 40

_mesh = plsc.VectorSubcoreMesh(core_axis_name="c", subcore_axis_name="s")


@functools.partial(
    pl.kernel,
    mesh=_mesh,
    out_type=jax.ShapeDtypeStruct((BATCH, SEQ, D), jnp.float32),
    scratch_types=[
        pltpu.VMEM((CB, SEQ), jnp.int32),
        pltpu.VMEM((CB, SEQ, D), jnp.float32),
        pltpu.VMEM((SEQ, D), jnp.float32),
        pltpu.SemaphoreType.DMA,
    ],
    compiler_params=pltpu.CompilerParams(use_tc_tiling_on_sc=False),
)
def _emb_lookup(x_hbm, item_hbm, pos_hbm, out_hbm, idx_v, rows_v, pos_v, sem):
    wid = lax.axis_index("s") * NC + lax.axis_index("c")
    pltpu.sync_copy(pos_hbm, pos_v)
    base_b = wid * BPW

    def chunk_body(c, carry):
        b0 = pl.multiple_of(base_b + c * CB, CB)
        pltpu.sync_copy(x_hbm.at[pl.ds(b0, CB)], idx_v)
        copies = [
            pltpu.async_copy(item_hbm.at[idx_v.at[b, pl.ds(h * GW, GW)]],
                             rows_v.at[b, pl.ds(h * GW, GW)], sem)
            for b in range(CB)
            for h in range(SEQ // GW)
        ]
        for cp in copies:
            cp.wait()

        def row_body(r, rcarry):
            p0 = pos_v[r, pl.ds(0, 16)]
            p1 = pos_v[r, pl.ds(16, 16)]
            for b in range(CB):
                plsc.addupdate(rows_v.at[b, r, pl.ds(0, 16)], p0)
                plsc.addupdate(rows_v.at[b, r, pl.ds(16, 16)], p1)
            return rcarry

        lax.fori_loop(0, SEQ, row_body, 0)
        pltpu.sync_copy(rows_v, out_hbm.at[pl.ds(b0, CB)])
        return carry

    lax.fori_loop(0, NCHUNK, chunk_body, 0)


def kernel(x, item_emb, pos_emb):
    return _emb_lookup(x, item_emb, pos_emb)
